# baseline (speedup 1.0000x reference)
import functools

import jax
import jax.numpy as jnp
from jax import lax
from jax.experimental import pallas as pl
from jax.experimental.pallas import tpu as pltpu
from jax.experimental.pallas import tpu_sc as plsc

N = 10000
M = 32
D = 128
BD = 16
OUT_D = 2 * D
E = N * M

_NC = 2
_NS = 16
_NW = _NC * _NS

_S = 5
_SE = E // _S
_SN = N // _S
_PER_W = _SE // _NW
_CHUNK = 200
_NCHUNK = _PER_W // _CHUNK

_EPS = 1e-5


def _sc_gather(table, idx_flat):
    mesh = plsc.VectorSubcoreMesh(core_axis_name="c", subcore_axis_name="s")

    @functools.partial(
        pl.kernel,
        out_type=jax.ShapeDtypeStruct((_SE, D), jnp.float32),
        mesh=mesh,
        scratch_types=[
            pltpu.VMEM((_PER_W,), jnp.int32),
            pltpu.VMEM((_CHUNK, D), jnp.float32),
            pltpu.VMEM((_CHUNK, D), jnp.float32),
            pltpu.SemaphoreType.DMA,
            pltpu.SemaphoreType.DMA,
            pltpu.SemaphoreType.DMA,
        ],
    )
    def gather_kernel(table_hbm, idx_hbm, out_hbm, idx_v, rows0, rows1,
                      gsem, osem0, osem1):
        wid = lax.axis_index("s") * _NC + lax.axis_index("c")
        base = pl.multiple_of(wid * _PER_W, 8)
        rows = (rows0, rows1)
        osems = (osem0, osem1)

        pltpu.sync_copy(idx_hbm.at[pl.ds(base, _PER_W)], idx_v)

        def chunk(k_static_pair, j, b):
            k = j * 2 + b
            off = pl.multiple_of(base + k * _CHUNK, 8)
            loc = pl.multiple_of(k * _CHUNK, 8)
            pltpu.async_copy(
                table_hbm.at[idx_v.at[pl.ds(loc, _CHUNK)]], rows[b], gsem
            ).wait()
            pltpu.async_copy(rows[b], out_hbm.at[pl.ds(off, _CHUNK)],
                             osems[b])

        def drain(b):
            pltpu.make_async_copy(
                out_hbm.at[pl.ds(base, _CHUNK)], rows[b], osems[b]).wait()

        for b in range(2):
            chunk(b, 0, b)

        def body(j, carry):
            for b in range(2):
                drain(b)
                chunk(None, j, b)
            return carry

        lax.fori_loop(1, _NCHUNK // 2, body, 0)
        for b in range(2):
            drain(b)

    return gather_kernel(table, idx_flat)


def _bf16_dot(a, w):
    return jnp.dot(a, w, preferred_element_type=jnp.float32,
                   precision=lax.Precision.DEFAULT)


def _edges(bond_ref):
    return bond_ref[...].reshape(_BE, BD)


_NB = 200
_NBLK = N // _NB
_BE = _NB * M


def _edge_y(a_ref, ag_ref, bond_ref, ws_ref, wn_ref, wb_ref, b_ref):
    p = _bf16_dot(a_ref[...], ws_ref[...]) + b_ref[...]
    y = _bf16_dot(ag_ref[...], wn_ref[...])
    y = y + _bf16_dot(_edges(bond_ref), wb_ref[...])
    return y.reshape(_NB, M, OUT_D) + p[:, None, :]


def _pass1_body(a_ref, ag_ref, bond_ref, ws_ref, wn_ref, wb_ref, b_ref,
                stats_ref):
    i = pl.program_id(0)
    y3 = _edge_y(a_ref, ag_ref, bond_ref, ws_ref, wn_ref, wb_ref, b_ref)
    s = jnp.sum(y3, axis=(0, 1))
    ss = jnp.sum(y3 * y3, axis=(0, 1))

    @pl.when(i == 0)
    def _():
        stats_ref[...] = jnp.zeros_like(stats_ref)

    stats_ref[0:1, :] += s[None, :]
    stats_ref[1:2, :] += ss[None, :]


def _pass2_body(a_ref, ag_ref, bond_ref, ws_ref, wn_ref, wb_ref, b_ref,
                stats2_ref, g2_ref, bt2_ref, s_ref, stats1_ref):
    i = pl.program_id(0)
    inv_b = 1.0 / E
    st = jnp.sum(stats2_ref[...].reshape(_S, 8, OUT_D), axis=0)
    mean = st[0:1, :] * inv_b
    var = st[1:2, :] * inv_b - mean * mean
    inv = lax.rsqrt(var + _EPS)
    scale = g2_ref[...] * inv
    shift = bt2_ref[...] - mean * scale

    y3 = _edge_y(a_ref, ag_ref, bond_ref, ws_ref, wn_ref, wb_ref, b_ref)
    z = y3 * scale + shift
    filt = z[..., :D]
    core = z[..., D:]
    sig = 1.0 / (1.0 + jnp.exp(-filt))
    sp = jnp.maximum(core, 0.0) + jnp.log1p(jnp.exp(-jnp.abs(core)))
    s_blk = jnp.sum(sig * sp, axis=1)
    s_ref[...] = s_blk

    @pl.when(i == 0)
    def _():
        stats1_ref[...] = jnp.zeros_like(stats1_ref)

    stats1_ref[0:1, :] += jnp.sum(s_blk, axis=0)[None, :]
    stats1_ref[1:2, :] += jnp.sum(s_blk * s_blk, axis=0)[None, :]


def _final_body(atom_ref, s_ref, stats1_ref, g1_ref, bt1_ref, out_ref):
    inv_n = 1.0 / N
    st = jnp.sum(stats1_ref[...].reshape(_S, 8, D), axis=0)
    mean = st[0:1, :] * inv_n
    var = st[1:2, :] * inv_n - mean * mean
    inv = lax.rsqrt(var + _EPS)
    scale = g1_ref[...] * inv
    shift = bt1_ref[...] - mean * scale
    x = atom_ref[...] + s_ref[...] * scale + shift
    out_ref[...] = jnp.maximum(x, 0.0) + jnp.log1p(jnp.exp(-jnp.abs(x)))


def _tc_pipeline(atom_in_fea, a_gs, bond3, ws, wn, wb, b2, g2, bt2, g1, bt1):
    full = lambda shape: pl.BlockSpec(shape, lambda i: (0, 0))
    ag_spec = pl.BlockSpec((_BE, D), lambda i: (i, 0))
    w_specs = [full((D, OUT_D)), full((D, OUT_D)), full((BD, OUT_D)),
               full((1, OUT_D))]
    nblk = _SN // _NB

    def sliced(k):
        a_spec = pl.BlockSpec((_NB, D), lambda i, k=k: (i + nblk * k, 0))
        bond_spec = pl.BlockSpec((_NB, M, BD),
                                 lambda i, k=k: (i + nblk * k, 0, 0))
        return a_spec, bond_spec

    stats2_parts = []
    for k in range(_S):
        a_spec, bond_spec = sliced(k)
        stats2_parts.append(pl.pallas_call(
            _pass1_body,
            grid=(nblk,),
            in_specs=[a_spec, ag_spec, bond_spec] + w_specs,
            out_specs=full((8, OUT_D)),
            out_shape=jax.ShapeDtypeStruct((8, OUT_D), jnp.float32),
        )(atom_in_fea, a_gs[k], bond3, ws, wn, wb, b2))
    stats2 = jnp.stack(stats2_parts).reshape(_S * 8, OUT_D)

    s_parts, stats1_parts = [], []
    for k in range(_S):
        a_spec, bond_spec = sliced(k)
        s_k, st1_k = pl.pallas_call(
            _pass2_body,
            grid=(nblk,),
            in_specs=[a_spec, ag_spec, bond_spec] + w_specs
            + [full((_S * 8, OUT_D)), full((1, OUT_D)), full((1, OUT_D))],
            out_specs=[pl.BlockSpec((_NB, D), lambda i: (i, 0)),
                       full((8, D))],
            out_shape=[jax.ShapeDtypeStruct((_SN, D), jnp.float32),
                       jax.ShapeDtypeStruct((8, D), jnp.float32)],
        )(atom_in_fea, a_gs[k], bond3, ws, wn, wb, b2, stats2, g2, bt2)
        s_parts.append(s_k)
        stats1_parts.append(st1_k)
    stats1 = jnp.stack(stats1_parts).reshape(_S * 8, D)

    out_parts = []
    for k in range(_S):
        a_spec, _ = sliced(k)
        out_parts.append(pl.pallas_call(
            _final_body,
            grid=(1,),
            in_specs=[pl.BlockSpec((_SN, D), lambda i, k=k: (k, 0)),
                      pl.BlockSpec((_SN, D), lambda i: (0, 0)),
                      full((_S * 8, D)), full((1, D)), full((1, D))],
            out_specs=pl.BlockSpec((_SN, D), lambda i: (0, 0)),
            out_shape=jax.ShapeDtypeStruct((_SN, D), jnp.float32),
        )(atom_in_fea, s_parts[k], stats1, g1, bt1))
    return jnp.concatenate(out_parts, axis=0)


def kernel(atom_in_fea, bond_nbr_fea, nbr_fea_idx, W, b, gamma2, beta2,
           gamma1, beta1):
    idx_flat = nbr_fea_idx.reshape(-1).astype(jnp.int32)
    ws = W[:D, :]
    wn = W[D:2 * D, :]
    wb = W[2 * D:, :]
    b2 = b.reshape(1, OUT_D)
    g2 = gamma2.reshape(1, OUT_D)
    bt2 = beta2.reshape(1, OUT_D)
    g1 = gamma1.reshape(1, D)
    bt1 = beta1.reshape(1, D)

    a_gs = [_sc_gather(atom_in_fea, idx_flat[k * _SE:(k + 1) * _SE])
            for k in range(_S)]
    return _tc_pipeline(atom_in_fea, a_gs, bond_nbr_fea.astype(jnp.bfloat16),
                        ws, wn, wb, b2, g2, bt2, g1, bt1)


# speedup vs baseline: 1.0660x; 1.0660x over previous
import functools

import jax
import jax.numpy as jnp
from jax import lax
from jax.experimental import pallas as pl
from jax.experimental.pallas import tpu as pltpu
from jax.experimental.pallas import tpu_sc as plsc

N = 10000
M = 32
D = 128
BD = 16
OUT_D = 2 * D
E = N * M

_NC = 2
_NS = 16
_NW = _NC * _NS

_S = 1
_SE = E // _S
_SN = N // _S
_PER_W = _SE // _NW
_CHUNK = 200
_NCHUNK = _PER_W // _CHUNK

_EPS = 1e-5


def _sc_gather(table, idx_flat):
    mesh = plsc.VectorSubcoreMesh(core_axis_name="c", subcore_axis_name="s")

    @functools.partial(
        pl.kernel,
        out_type=jax.ShapeDtypeStruct((_SE, D), jnp.float32),
        mesh=mesh,
        scratch_types=[
            pltpu.VMEM((_PER_W,), jnp.int32),
            pltpu.VMEM((_CHUNK, D), jnp.float32),
            pltpu.VMEM((_CHUNK, D), jnp.float32),
            pltpu.SemaphoreType.DMA,
            pltpu.SemaphoreType.DMA,
            pltpu.SemaphoreType.DMA,
        ],
    )
    def gather_kernel(table_hbm, idx_hbm, out_hbm, idx_v, rows0, rows1,
                      gsem, osem0, osem1):
        wid = lax.axis_index("s") * _NC + lax.axis_index("c")
        base = pl.multiple_of(wid * _PER_W, 8)
        rows = (rows0, rows1)
        osems = (osem0, osem1)

        pltpu.sync_copy(idx_hbm.at[pl.ds(base, _PER_W)], idx_v)

        def chunk(k_static_pair, j, b):
            k = j * 2 + b
            off = pl.multiple_of(base + k * _CHUNK, 8)
            loc = pl.multiple_of(k * _CHUNK, 8)
            pltpu.async_copy(
                table_hbm.at[idx_v.at[pl.ds(loc, _CHUNK)]], rows[b], gsem
            ).wait()
            pltpu.async_copy(rows[b], out_hbm.at[pl.ds(off, _CHUNK)],
                             osems[b])

        def drain(b):
            pltpu.make_async_copy(
                out_hbm.at[pl.ds(base, _CHUNK)], rows[b], osems[b]).wait()

        for b in range(2):
            chunk(b, 0, b)

        def body(j, carry):
            for b in range(2):
                drain(b)
                chunk(None, j, b)
            return carry

        lax.fori_loop(1, _NCHUNK // 2, body, 0)
        for b in range(2):
            drain(b)

    return gather_kernel(table, idx_flat)


def _bf16_dot(a, w):
    return jnp.dot(a, w, preferred_element_type=jnp.float32,
                   precision=lax.Precision.DEFAULT)


def _edges(bond_ref):
    return bond_ref[...].reshape(_BE, BD)


_NB = 200
_NBLK = N // _NB
_BE = _NB * M


def _edge_y(a_ref, ag_ref, bond_ref, ws_ref, wn_ref, wb_ref, b_ref):
    p = _bf16_dot(a_ref[...], ws_ref[...]) + b_ref[...]
    y = _bf16_dot(ag_ref[...], wn_ref[...])
    y = y + _bf16_dot(_edges(bond_ref), wb_ref[...])
    return y.reshape(_NB, M, OUT_D) + p[:, None, :]


def _pass1_body(a_ref, ag_ref, bond_ref, ws_ref, wn_ref, wb_ref, b_ref,
                stats_ref):
    i = pl.program_id(0)
    y3 = _edge_y(a_ref, ag_ref, bond_ref, ws_ref, wn_ref, wb_ref, b_ref)
    s = jnp.sum(y3, axis=(0, 1))
    ss = jnp.sum(y3 * y3, axis=(0, 1))

    @pl.when(i == 0)
    def _():
        stats_ref[...] = jnp.zeros_like(stats_ref)

    stats_ref[0:1, :] += s[None, :]
    stats_ref[1:2, :] += ss[None, :]


def _pass2_body(a_ref, ag_ref, bond_ref, ws_ref, wn_ref, wb_ref, b_ref,
                stats2_ref, g2_ref, bt2_ref, s_ref, stats1_ref):
    i = pl.program_id(0)
    inv_b = 1.0 / E
    st = jnp.sum(stats2_ref[...].reshape(_S, 8, OUT_D), axis=0)
    mean = st[0:1, :] * inv_b
    var = st[1:2, :] * inv_b - mean * mean
    inv = lax.rsqrt(var + _EPS)
    scale = g2_ref[...] * inv
    shift = bt2_ref[...] - mean * scale

    y3 = _edge_y(a_ref, ag_ref, bond_ref, ws_ref, wn_ref, wb_ref, b_ref)
    z = y3 * scale + shift
    filt = z[..., :D]
    core = z[..., D:]
    sig = 1.0 / (1.0 + jnp.exp(-filt))
    sp = jnp.maximum(core, 0.0) + jnp.log1p(jnp.exp(-jnp.abs(core)))
    s_blk = jnp.sum(sig * sp, axis=1)
    s_ref[...] = s_blk

    @pl.when(i == 0)
    def _():
        stats1_ref[...] = jnp.zeros_like(stats1_ref)

    stats1_ref[0:1, :] += jnp.sum(s_blk, axis=0)[None, :]
    stats1_ref[1:2, :] += jnp.sum(s_blk * s_blk, axis=0)[None, :]


def _final_body(atom_ref, s_ref, stats1_ref, g1_ref, bt1_ref, out_ref):
    inv_n = 1.0 / N
    st = jnp.sum(stats1_ref[...].reshape(_S, 8, D), axis=0)
    mean = st[0:1, :] * inv_n
    var = st[1:2, :] * inv_n - mean * mean
    inv = lax.rsqrt(var + _EPS)
    scale = g1_ref[...] * inv
    shift = bt1_ref[...] - mean * scale
    x = atom_ref[...] + s_ref[...] * scale + shift
    out_ref[...] = jnp.maximum(x, 0.0) + jnp.log1p(jnp.exp(-jnp.abs(x)))


def _tc_pipeline(atom_in_fea, a_gs, bond3, ws, wn, wb, b2, g2, bt2, g1, bt1):
    full = lambda shape: pl.BlockSpec(shape, lambda i: (0, 0))
    ag_spec = pl.BlockSpec((_BE, D), lambda i: (i, 0))
    w_specs = [full((D, OUT_D)), full((D, OUT_D)), full((BD, OUT_D)),
               full((1, OUT_D))]
    nblk = _SN // _NB

    def sliced(k):
        a_spec = pl.BlockSpec((_NB, D), lambda i, k=k: (i + nblk * k, 0))
        bond_spec = pl.BlockSpec((_NB, M, BD),
                                 lambda i, k=k: (i + nblk * k, 0, 0))
        return a_spec, bond_spec

    stats2_parts = []
    for k in range(_S):
        a_spec, bond_spec = sliced(k)
        stats2_parts.append(pl.pallas_call(
            _pass1_body,
            grid=(nblk,),
            in_specs=[a_spec, ag_spec, bond_spec] + w_specs,
            out_specs=full((8, OUT_D)),
            out_shape=jax.ShapeDtypeStruct((8, OUT_D), jnp.float32),
        )(atom_in_fea, a_gs[k], bond3, ws, wn, wb, b2))
    stats2 = jnp.stack(stats2_parts).reshape(_S * 8, OUT_D)

    s_parts, stats1_parts = [], []
    for k in range(_S):
        a_spec, bond_spec = sliced(k)
        s_k, st1_k = pl.pallas_call(
            _pass2_body,
            grid=(nblk,),
            in_specs=[a_spec, ag_spec, bond_spec] + w_specs
            + [full((_S * 8, OUT_D)), full((1, OUT_D)), full((1, OUT_D))],
            out_specs=[pl.BlockSpec((_NB, D), lambda i: (i, 0)),
                       full((8, D))],
            out_shape=[jax.ShapeDtypeStruct((_SN, D), jnp.float32),
                       jax.ShapeDtypeStruct((8, D), jnp.float32)],
        )(atom_in_fea, a_gs[k], bond3, ws, wn, wb, b2, stats2, g2, bt2)
        s_parts.append(s_k)
        stats1_parts.append(st1_k)
    stats1 = jnp.stack(stats1_parts).reshape(_S * 8, D)

    out_parts = []
    for k in range(_S):
        a_spec, _ = sliced(k)
        out_parts.append(pl.pallas_call(
            _final_body,
            grid=(1,),
            in_specs=[pl.BlockSpec((_SN, D), lambda i, k=k: (k, 0)),
                      pl.BlockSpec((_SN, D), lambda i: (0, 0)),
                      full((_S * 8, D)), full((1, D)), full((1, D))],
            out_specs=pl.BlockSpec((_SN, D), lambda i: (0, 0)),
            out_shape=jax.ShapeDtypeStruct((_SN, D), jnp.float32),
        )(atom_in_fea, s_parts[k], stats1, g1, bt1))
    return jnp.concatenate(out_parts, axis=0)


def kernel(atom_in_fea, bond_nbr_fea, nbr_fea_idx, W, b, gamma2, beta2,
           gamma1, beta1):
    idx_flat = nbr_fea_idx.reshape(-1).astype(jnp.int32)
    ws = W[:D, :]
    wn = W[D:2 * D, :]
    wb = W[2 * D:, :]
    b2 = b.reshape(1, OUT_D)
    g2 = gamma2.reshape(1, OUT_D)
    bt2 = beta2.reshape(1, OUT_D)
    g1 = gamma1.reshape(1, D)
    bt1 = beta1.reshape(1, D)

    a_gs = [_sc_gather(atom_in_fea, idx_flat[k * _SE:(k + 1) * _SE])
            for k in range(_S)]
    return _tc_pipeline(atom_in_fea, a_gs, bond_nbr_fea.astype(jnp.bfloat16),
                        ws, wn, wb, b2, g2, bt2, g1, bt1)


# speedup vs baseline: 1.0711x; 1.0048x over previous
import functools

import jax
import jax.numpy as jnp
from jax import lax
from jax.experimental import pallas as pl
from jax.experimental.pallas import tpu as pltpu
from jax.experimental.pallas import tpu_sc as plsc

N = 10000
M = 32
D = 128
BD = 16
OUT_D = 2 * D
E = N * M

_NC = 2
_NS = 16
_NW = _NC * _NS

_S = 1
_SE = E // _S
_SN = N // _S
_PER_W = _SE // _NW
_CHUNK = 200
_NCHUNK = _PER_W // _CHUNK

_EPS = 1e-5


def _sc_gather(table, idx_flat):
    mesh = plsc.VectorSubcoreMesh(core_axis_name="c", subcore_axis_name="s")

    @functools.partial(
        pl.kernel,
        out_type=jax.ShapeDtypeStruct((_SE, D), jnp.float32),
        mesh=mesh,
        scratch_types=[pltpu.VMEM((_PER_W,), jnp.int32)]
        + [pltpu.VMEM((_CHUNK, D), jnp.float32)] * 4
        + [pltpu.SemaphoreType.DMA] * 8,
    )
    def gather_kernel(table_hbm, idx_hbm, out_hbm, idx_v, r0, r1, r2, r3,
                      g0, g1, g2, g3, o0, o1, o2, o3):
        wid = lax.axis_index("s") * _NC + lax.axis_index("c")
        base = pl.multiple_of(wid * _PER_W, 8)
        rows = (r0, r1, r2, r3)
        gsems = (g0, g1, g2, g3)
        osems = (o0, o1, o2, o3)

        pltpu.sync_copy(idx_hbm.at[pl.ds(base, _PER_W)], idx_v)

        def issue_gather(k, b):
            loc = pl.multiple_of(k * _CHUNK, 8)
            pltpu.async_copy(
                table_hbm.at[idx_v.at[pl.ds(loc, _CHUNK)]], rows[b],
                gsems[b])

        def wait_gather(b):
            pltpu.make_async_copy(
                table_hbm.at[pl.ds(0, _CHUNK)], rows[b], gsems[b]).wait()

        def start_out(k, b):
            off = pl.multiple_of(base + k * _CHUNK, 8)
            pltpu.async_copy(rows[b], out_hbm.at[pl.ds(off, _CHUNK)],
                             osems[b])

        def drain_out(b):
            pltpu.make_async_copy(
                out_hbm.at[pl.ds(base, _CHUNK)], rows[b], osems[b]).wait()

        def step(k, b, do_drain, do_issue):
            wait_gather(b)
            start_out(k, b)
            if do_issue:
                b2 = (b + 2) % 4
                if do_drain:
                    drain_out(b2)
                issue_gather(k + 2, b2)

        issue_gather(0, 0)
        issue_gather(1, 1)
        step(0, 0, False, True)
        step(1, 1, False, True)
        step(2, 2, True, True)
        step(3, 3, True, True)

        def body(j, carry):
            for b in range(4):
                step(j * 4 + b, b, True, True)
            return carry

        lax.fori_loop(1, _NCHUNK // 4, body, 0)

        step(_NCHUNK - 2, 0, False, False)
        step(_NCHUNK - 1, 1, False, False)
        for b in (2, 3, 0, 1):
            drain_out(b)

    return gather_kernel(table, idx_flat)


def _bf16_dot(a, w):
    return jnp.dot(a, w, preferred_element_type=jnp.float32,
                   precision=lax.Precision.DEFAULT)


def _edges(bond_ref):
    return bond_ref[...].reshape(_BE, BD)


_NB = 200
_NBLK = N // _NB
_BE = _NB * M


def _edge_y(a_ref, ag_ref, bond_ref, ws_ref, wn_ref, wb_ref, b_ref):
    p = _bf16_dot(a_ref[...], ws_ref[...]) + b_ref[...]
    y = _bf16_dot(ag_ref[...], wn_ref[...])
    y = y + _bf16_dot(_edges(bond_ref), wb_ref[...])
    return y.reshape(_NB, M, OUT_D) + p[:, None, :]


def _pass1_body(a_ref, ag_ref, bond_ref, ws_ref, wn_ref, wb_ref, b_ref,
                stats_ref):
    i = pl.program_id(0)
    y3 = _edge_y(a_ref, ag_ref, bond_ref, ws_ref, wn_ref, wb_ref, b_ref)
    s = jnp.sum(y3, axis=(0, 1))
    ss = jnp.sum(y3 * y3, axis=(0, 1))

    @pl.when(i == 0)
    def _():
        stats_ref[...] = jnp.zeros_like(stats_ref)

    stats_ref[0:1, :] += s[None, :]
    stats_ref[1:2, :] += ss[None, :]


def _pass2_body(a_ref, ag_ref, bond_ref, ws_ref, wn_ref, wb_ref, b_ref,
                stats2_ref, g2_ref, bt2_ref, s_ref, stats1_ref):
    i = pl.program_id(0)
    inv_b = 1.0 / E
    st = jnp.sum(stats2_ref[...].reshape(_S, 8, OUT_D), axis=0)
    mean = st[0:1, :] * inv_b
    var = st[1:2, :] * inv_b - mean * mean
    inv = lax.rsqrt(var + _EPS)
    scale = g2_ref[...] * inv
    shift = bt2_ref[...] - mean * scale

    y3 = _edge_y(a_ref, ag_ref, bond_ref, ws_ref, wn_ref, wb_ref, b_ref)
    z = y3 * scale + shift
    filt = z[..., :D]
    core = z[..., D:]
    sig = 1.0 / (1.0 + jnp.exp(-filt))
    sp = jnp.maximum(core, 0.0) + jnp.log1p(jnp.exp(-jnp.abs(core)))
    s_blk = jnp.sum(sig * sp, axis=1)
    s_ref[...] = s_blk

    @pl.when(i == 0)
    def _():
        stats1_ref[...] = jnp.zeros_like(stats1_ref)

    stats1_ref[0:1, :] += jnp.sum(s_blk, axis=0)[None, :]
    stats1_ref[1:2, :] += jnp.sum(s_blk * s_blk, axis=0)[None, :]


def _final_body(atom_ref, s_ref, stats1_ref, g1_ref, bt1_ref, out_ref):
    inv_n = 1.0 / N
    st = jnp.sum(stats1_ref[...].reshape(_S, 8, D), axis=0)
    mean = st[0:1, :] * inv_n
    var = st[1:2, :] * inv_n - mean * mean
    inv = lax.rsqrt(var + _EPS)
    scale = g1_ref[...] * inv
    shift = bt1_ref[...] - mean * scale
    x = atom_ref[...] + s_ref[...] * scale + shift
    out_ref[...] = jnp.maximum(x, 0.0) + jnp.log1p(jnp.exp(-jnp.abs(x)))


def _tc_pipeline(atom_in_fea, a_gs, bond3, ws, wn, wb, b2, g2, bt2, g1, bt1):
    full = lambda shape: pl.BlockSpec(shape, lambda i: (0, 0))
    ag_spec = pl.BlockSpec((_BE, D), lambda i: (i, 0))
    w_specs = [full((D, OUT_D)), full((D, OUT_D)), full((BD, OUT_D)),
               full((1, OUT_D))]
    nblk = _SN // _NB

    def sliced(k):
        a_spec = pl.BlockSpec((_NB, D), lambda i, k=k: (i + nblk * k, 0))
        bond_spec = pl.BlockSpec((_NB, M, BD),
                                 lambda i, k=k: (i + nblk * k, 0, 0))
        return a_spec, bond_spec

    stats2_parts = []
    for k in range(_S):
        a_spec, bond_spec = sliced(k)
        stats2_parts.append(pl.pallas_call(
            _pass1_body,
            grid=(nblk,),
            in_specs=[a_spec, ag_spec, bond_spec] + w_specs,
            out_specs=full((8, OUT_D)),
            out_shape=jax.ShapeDtypeStruct((8, OUT_D), jnp.float32),
        )(atom_in_fea, a_gs[k], bond3, ws, wn, wb, b2))
    stats2 = jnp.stack(stats2_parts).reshape(_S * 8, OUT_D)

    s_parts, stats1_parts = [], []
    for k in range(_S):
        a_spec, bond_spec = sliced(k)
        s_k, st1_k = pl.pallas_call(
            _pass2_body,
            grid=(nblk,),
            in_specs=[a_spec, ag_spec, bond_spec] + w_specs
            + [full((_S * 8, OUT_D)), full((1, OUT_D)), full((1, OUT_D))],
            out_specs=[pl.BlockSpec((_NB, D), lambda i: (i, 0)),
                       full((8, D))],
            out_shape=[jax.ShapeDtypeStruct((_SN, D), jnp.float32),
                       jax.ShapeDtypeStruct((8, D), jnp.float32)],
        )(atom_in_fea, a_gs[k], bond3, ws, wn, wb, b2, stats2, g2, bt2)
        s_parts.append(s_k)
        stats1_parts.append(st1_k)
    stats1 = jnp.stack(stats1_parts).reshape(_S * 8, D)

    out_parts = []
    for k in range(_S):
        a_spec, _ = sliced(k)
        out_parts.append(pl.pallas_call(
            _final_body,
            grid=(1,),
            in_specs=[pl.BlockSpec((_SN, D), lambda i, k=k: (k, 0)),
                      pl.BlockSpec((_SN, D), lambda i: (0, 0)),
                      full((_S * 8, D)), full((1, D)), full((1, D))],
            out_specs=pl.BlockSpec((_SN, D), lambda i: (0, 0)),
            out_shape=jax.ShapeDtypeStruct((_SN, D), jnp.float32),
        )(atom_in_fea, s_parts[k], stats1, g1, bt1))
    return jnp.concatenate(out_parts, axis=0)


def kernel(atom_in_fea, bond_nbr_fea, nbr_fea_idx, W, b, gamma2, beta2,
           gamma1, beta1):
    idx_flat = nbr_fea_idx.reshape(-1).astype(jnp.int32)
    ws = W[:D, :]
    wn = W[D:2 * D, :]
    wb = W[2 * D:, :]
    b2 = b.reshape(1, OUT_D)
    g2 = gamma2.reshape(1, OUT_D)
    bt2 = beta2.reshape(1, OUT_D)
    g1 = gamma1.reshape(1, D)
    bt1 = beta1.reshape(1, D)

    a_gs = [_sc_gather(atom_in_fea, idx_flat[k * _SE:(k + 1) * _SE])
            for k in range(_S)]
    return _tc_pipeline(atom_in_fea, a_gs, bond_nbr_fea.astype(jnp.bfloat16),
                        ws, wn, wb, b2, g2, bt2, g1, bt1)


# speedup vs baseline: 1.1051x; 1.0318x over previous
import functools

import jax
import jax.numpy as jnp
from jax import lax
from jax.experimental import pallas as pl
from jax.experimental.pallas import tpu as pltpu
from jax.experimental.pallas import tpu_sc as plsc

N = 10000
M = 32
D = 128
BD = 16
OUT_D = 2 * D
E = N * M

_NC = 2
_NS = 16
_NW = _NC * _NS

_S = 1
_SE = E // _S
_SN = N // _S
_PER_W = _SE // _NW
_CHUNK = 200
_NCHUNK = _PER_W // _CHUNK

_EPS = 1e-5


def _sc_gather(table, idx_flat):
    mesh = plsc.VectorSubcoreMesh(core_axis_name="c", subcore_axis_name="s")

    @functools.partial(
        pl.kernel,
        out_type=jax.ShapeDtypeStruct((_SE, D), jnp.float32),
        mesh=mesh,
        scratch_types=[pltpu.VMEM((_PER_W,), jnp.int32)]
        + [pltpu.VMEM((_CHUNK, D), jnp.float32)] * 4
        + [pltpu.SemaphoreType.DMA] * 8,
    )
    def gather_kernel(table_hbm, idx_hbm, out_hbm, idx_v, r0, r1, r2, r3,
                      g0, g1, g2, g3, o0, o1, o2, o3):
        wid = lax.axis_index("s") * _NC + lax.axis_index("c")
        base = pl.multiple_of(wid * _PER_W, 8)
        rows = (r0, r1, r2, r3)
        gsems = (g0, g1, g2, g3)
        osems = (o0, o1, o2, o3)

        pltpu.sync_copy(idx_hbm.at[pl.ds(base, _PER_W)], idx_v)

        def issue_gather(k, b):
            loc = pl.multiple_of(k * _CHUNK, 8)
            pltpu.async_copy(
                table_hbm.at[idx_v.at[pl.ds(loc, _CHUNK)]], rows[b],
                gsems[b])

        def wait_gather(b):
            pltpu.make_async_copy(
                table_hbm.at[pl.ds(0, _CHUNK)], rows[b], gsems[b]).wait()

        def start_out(k, b):
            off = pl.multiple_of(base + k * _CHUNK, 8)
            pltpu.async_copy(rows[b], out_hbm.at[pl.ds(off, _CHUNK)],
                             osems[b])

        def drain_out(b):
            pltpu.make_async_copy(
                out_hbm.at[pl.ds(base, _CHUNK)], rows[b], osems[b]).wait()

        def step(k, b, do_drain, do_issue):
            wait_gather(b)
            start_out(k, b)
            if do_issue:
                b2 = (b + 2) % 4
                if do_drain:
                    drain_out(b2)
                issue_gather(k + 2, b2)

        issue_gather(0, 0)
        issue_gather(1, 1)
        step(0, 0, False, True)
        step(1, 1, False, True)
        step(2, 2, True, True)
        step(3, 3, True, True)

        def body(j, carry):
            for b in range(4):
                step(j * 4 + b, b, True, True)
            return carry

        lax.fori_loop(1, _NCHUNK // 4, body, 0)

        step(_NCHUNK - 2, 0, False, False)
        step(_NCHUNK - 1, 1, False, False)
        for b in (2, 3, 0, 1):
            drain_out(b)

    return gather_kernel(table, idx_flat)


def _bf16_dot(a, w):
    return jnp.dot(a, w, preferred_element_type=jnp.float32,
                   precision=lax.Precision.DEFAULT)


def _edges(bond_ref):
    return bond_ref[...].reshape(_BE, BD)


_NB = 400
_NBLK = N // _NB
_BE = _NB * M


def _edge_y(a_ref, ag_ref, bond_ref, ws_ref, wn_ref, wb_ref, b_ref):
    p = _bf16_dot(a_ref[...], ws_ref[...]) + b_ref[...]
    y = _bf16_dot(ag_ref[...], wn_ref[...])
    y = y + _bf16_dot(_edges(bond_ref), wb_ref[...])
    return y.reshape(_NB, M, OUT_D) + p[:, None, :]


def _pass1_body(a_ref, ag_ref, bond_ref, ws_ref, wn_ref, wb_ref, b_ref,
                stats_ref):
    i = pl.program_id(0)
    y3 = _edge_y(a_ref, ag_ref, bond_ref, ws_ref, wn_ref, wb_ref, b_ref)
    s = jnp.sum(y3, axis=(0, 1))
    ss = jnp.sum(y3 * y3, axis=(0, 1))

    @pl.when(i == 0)
    def _():
        stats_ref[...] = jnp.zeros_like(stats_ref)

    stats_ref[0:1, :] += s[None, :]
    stats_ref[1:2, :] += ss[None, :]


def _pass2_body(a_ref, ag_ref, bond_ref, ws_ref, wn_ref, wb_ref, b_ref,
                stats2_ref, g2_ref, bt2_ref, s_ref, stats1_ref):
    i = pl.program_id(0)
    inv_b = 1.0 / E
    st = jnp.sum(stats2_ref[...].reshape(_S, 8, OUT_D), axis=0)
    mean = st[0:1, :] * inv_b
    var = st[1:2, :] * inv_b - mean * mean
    inv = lax.rsqrt(var + _EPS)
    scale = g2_ref[...] * inv
    shift = bt2_ref[...] - mean * scale

    y3 = _edge_y(a_ref, ag_ref, bond_ref, ws_ref, wn_ref, wb_ref, b_ref)
    z = y3 * scale + shift
    filt = z[..., :D]
    core = z[..., D:]
    sig = 1.0 / (1.0 + jnp.exp(-filt))
    sp = jnp.maximum(core, 0.0) + jnp.log1p(jnp.exp(-jnp.abs(core)))
    s_blk = jnp.sum(sig * sp, axis=1)
    s_ref[...] = s_blk

    @pl.when(i == 0)
    def _():
        stats1_ref[...] = jnp.zeros_like(stats1_ref)

    stats1_ref[0:1, :] += jnp.sum(s_blk, axis=0)[None, :]
    stats1_ref[1:2, :] += jnp.sum(s_blk * s_blk, axis=0)[None, :]


def _final_body(atom_ref, s_ref, stats1_ref, g1_ref, bt1_ref, out_ref):
    inv_n = 1.0 / N
    st = jnp.sum(stats1_ref[...].reshape(_S, 8, D), axis=0)
    mean = st[0:1, :] * inv_n
    var = st[1:2, :] * inv_n - mean * mean
    inv = lax.rsqrt(var + _EPS)
    scale = g1_ref[...] * inv
    shift = bt1_ref[...] - mean * scale
    x = atom_ref[...] + s_ref[...] * scale + shift
    out_ref[...] = jnp.maximum(x, 0.0) + jnp.log1p(jnp.exp(-jnp.abs(x)))


def _tc_pipeline(atom_in_fea, a_gs, bond3, ws, wn, wb, b2, g2, bt2, g1, bt1):
    a_g = a_gs[0]
    full = lambda shape: pl.BlockSpec(shape, lambda i: (0, 0))
    a_spec = pl.BlockSpec((_NB, D), lambda i: (i, 0))
    ag_spec = pl.BlockSpec((_BE, D), lambda i: (i, 0))
    bond_spec = pl.BlockSpec((_NB, M, BD), lambda i: (i, 0, 0))
    w_specs = [full((D, OUT_D)), full((D, OUT_D)), full((BD, OUT_D)),
               full((1, OUT_D))]

    stats2 = pl.pallas_call(
        _pass1_body,
        grid=(_NBLK,),
        in_specs=[a_spec, ag_spec, bond_spec] + w_specs,
        out_specs=full((8, OUT_D)),
        out_shape=jax.ShapeDtypeStruct((8, OUT_D), jnp.float32),
    )(atom_in_fea, a_g, bond3, ws, wn, wb, b2)

    s, stats1 = pl.pallas_call(
        _pass2_body,
        grid=(_NBLK,),
        in_specs=[a_spec, ag_spec, bond_spec] + w_specs
        + [full((_S * 8, OUT_D)), full((1, OUT_D)), full((1, OUT_D))],
        out_specs=[pl.BlockSpec((_NB, D), lambda i: (i, 0)), full((8, D))],
        out_shape=[jax.ShapeDtypeStruct((N, D), jnp.float32),
                   jax.ShapeDtypeStruct((8, D), jnp.float32)],
    )(atom_in_fea, a_g, bond3, ws, wn, wb, b2, stats2, g2, bt2)

    _FB = 2000
    out = pl.pallas_call(
        _final_body,
        grid=(N // _FB,),
        in_specs=[pl.BlockSpec((_FB, D), lambda i: (i, 0)),
                  pl.BlockSpec((_FB, D), lambda i: (i, 0)),
                  full((_S * 8, D)), full((1, D)), full((1, D))],
        out_specs=pl.BlockSpec((_FB, D), lambda i: (i, 0)),
        out_shape=jax.ShapeDtypeStruct((N, D), jnp.float32),
    )(atom_in_fea, s, stats1, g1, bt1)
    return out


def kernel(atom_in_fea, bond_nbr_fea, nbr_fea_idx, W, b, gamma2, beta2,
           gamma1, beta1):
    idx_flat = nbr_fea_idx.reshape(-1).astype(jnp.int32)
    ws = W[:D, :]
    wn = W[D:2 * D, :]
    wb = W[2 * D:, :]
    b2 = b.reshape(1, OUT_D)
    g2 = gamma2.reshape(1, OUT_D)
    bt2 = beta2.reshape(1, OUT_D)
    g1 = gamma1.reshape(1, D)
    bt1 = beta1.reshape(1, D)

    a_gs = [_sc_gather(atom_in_fea, idx_flat[k * _SE:(k + 1) * _SE])
            for k in range(_S)]
    return _tc_pipeline(atom_in_fea, a_gs, bond_nbr_fea.astype(jnp.bfloat16),
                        ws, wn, wb, b2, g2, bt2, g1, bt1)
